# Initial kernel scaffold; baseline (speedup 1.0000x reference)
import jax
import jax.numpy as jnp
from jax.experimental import pallas as pl


def kernel(c1_w, c1_b, c2_w, c2_b, fc1_w, fc1_b, fc2_w, fc2_b, fc3_w, fc3_b, x):
    raise NotImplementedError("write your pallas kernel here")


# speedup vs baseline: 3.6359x; 3.6359x over previous
import numpy as np

import jax
import jax.numpy as jnp
from jax.experimental import pallas as pl
from jax.experimental.pallas import tpu as pltpu


_A = np.zeros((2, 2, 2, 4, 5), np.float32)
for _pe in range(2):
    for _p in range(2):
        for _q in range(2):
            for _rq in range(4):
                _di = 4 * _q + _rq - 2 * _pe - _p
                if 0 <= _di < 5:
                    _A[_pe, _p, _q, _rq, _di] = 1.0
_B = np.zeros((32, 2, 14, 5), np.float32)
for _br in range(2):
    for _col2 in range(14):
        for _dj in range(5):
            _ci = 2 * _col2 + _br + _dj
            if _ci < 32:
                _B[_ci, _br, _col2, _dj] = 1.0


def _build_conv1_weights(c1_w):
    w4 = c1_w[:, :15, :6].reshape(5, 5, 3, 6)
    e = jnp.einsum("PpQRD,CbKJ,DJco->PpcQRCbKo", _A, _B, w4)
    e = jnp.pad(e, ((0, 0),) * 7 + ((0, 0), (0, 2)))
    e = e.reshape(2, 2, 3, 256, 2, 112)
    e = jnp.pad(e, ((0, 0),) * 5 + ((0, 16),))
    return e.reshape(12, 256, 256).astype(jnp.bfloat16)


def _build_conv2_weights(c2_w):
    z = jnp.zeros((128, 256), c2_w.dtype)
    w = jnp.stack([
        jnp.concatenate([c2_w[0], c2_w[1]], axis=0),
        jnp.concatenate([c2_w[2], c2_w[3]], axis=0),
        jnp.concatenate([c2_w[4], z], axis=0),
        jnp.concatenate([z, c2_w[0]], axis=0),
        jnp.concatenate([c2_w[1], c2_w[2]], axis=0),
        jnp.concatenate([c2_w[3], c2_w[4]], axis=0),
    ])
    return w.astype(jnp.bfloat16)


def _build_fc1_weights(fc1_w):
    def ch(r):
        return jnp.pad(fc1_w[80 * r:80 * r + 80], ((0, 48), (0, 0)))
    z = jnp.zeros((128, 128), fc1_w.dtype)
    w = jnp.stack([
        jnp.concatenate([ch(0), ch(1)], axis=0),
        jnp.concatenate([ch(2), ch(3)], axis=0),
        jnp.concatenate([ch(4), z], axis=0),
    ])
    return w.astype(jnp.bfloat16)


def _fused_kernel(x_ref, w1_ref, b1_ref, w2_ref, b2_ref, wf_ref, bf1_ref,
                  w2f_ref, bf2_ref, w3f_ref, bf3_ref, o_ref):
    bt = x_ref.shape[0]
    R = bt * 8
    f32 = jnp.float32
    bf16 = jnp.bfloat16

    lhs = []
    for c in range(3):
        xc = x_ref[:, c].reshape(R, 128).astype(bf16)
        lhs.append(jnp.concatenate([xc[:R - 1], xc[1:]], axis=1))
    halves = []
    for pe in range(2):
        ms = None
        for p in range(2):
            acc = None
            for c in range(3):
                d = jnp.dot(lhs[c], w1_ref[(pe * 2 + p) * 3 + c],
                            preferred_element_type=f32)
                acc = d if acc is None else acc + d
            ms = acc if ms is None else jnp.maximum(ms, acc)
        m = jnp.maximum(ms[:, :128], ms[:, 128:])
        halves.append(jnp.maximum(m + b1_ref[...], 0.0))
    h1 = jnp.concatenate(halves, axis=1).astype(bf16)
    h1 = jnp.concatenate([h1, jnp.zeros((1, 256), bf16)], axis=0)

    accs = [None, None]
    for o in range(3):
        sl = h1[o:R - 2 + o]
        for p in range(2):
            d = jnp.dot(sl, w2_ref[p * 3 + o], preferred_element_type=f32)
            accs[p] = d if accs[p] is None else accs[p] + d
    m2 = jnp.maximum(accs[0], accs[1])
    m2 = jnp.maximum(m2[:, :128], m2[:, 128:])
    h2 = jnp.maximum(m2 + b2_ref[...], 0.0).astype(bf16)
    h2 = jnp.concatenate([h2, jnp.zeros((2, 128), bf16)], axis=0)
    h2 = h2.reshape(bt, 8, 128)

    f = None
    for r in range(3):
        l = jnp.concatenate([h2[:, 2 * r, :], h2[:, 2 * r + 1, :]], axis=1)
        d = jnp.dot(l, wf_ref[r], preferred_element_type=f32)
        f = d if f is None else f + d
    h = jnp.maximum(f + bf1_ref[...], 0.0).astype(bf16)
    h = jnp.dot(h, w2f_ref[...], preferred_element_type=f32)
    h = jnp.maximum(h + bf2_ref[...], 0.0).astype(bf16)
    out = jnp.dot(h, w3f_ref[...], preferred_element_type=f32) + bf3_ref[...]
    o_ref[...] = out


def kernel(c1_w, c1_b, c2_w, c2_b, fc1_w, fc1_b, fc2_w, fc2_b, fc3_w, fc3_b,
           x):
    B = x.shape[0]
    bt = next(b for b in (128, 64, 32, 16, 8, 4, 2, 1) if B % b == 0)
    x4 = x.reshape(B, 3, 8, 128)

    w1 = _build_conv1_weights(c1_w)
    w2 = _build_conv2_weights(c2_w)
    wf = _build_fc1_weights(fc1_w)
    w2f = fc2_w.astype(jnp.bfloat16)
    w3f = fc3_w.astype(jnp.bfloat16)

    full = lambda s: pl.BlockSpec(s, lambda i: (0,) * len(s))
    out = pl.pallas_call(
        _fused_kernel,
        out_shape=jax.ShapeDtypeStruct((B, 128), jnp.float32),
        grid=(B // bt,),
        in_specs=[
            pl.BlockSpec((bt, 3, 8, 128), lambda i: (i, 0, 0, 0)),
            full((12, 256, 256)), full((1, 128)),
            full((6, 256, 256)), full((1, 128)),
            full((3, 256, 128)), full((1, 128)),
            full((128, 128)), full((1, 128)),
            full((128, 128)), full((1, 128)),
        ],
        out_specs=pl.BlockSpec((bt, 128), lambda i: (i, 0)),
        compiler_params=pltpu.CompilerParams(
            dimension_semantics=("parallel",),
            vmem_limit_bytes=48 * 1024 * 1024),
    )(x4, w1, c1_b, w2, c2_b, wf, fc1_b, w2f, fc2_b, w3f, fc3_b)
    return out[:, :10]
